# baseline (speedup 1.0000x reference)
import jax
import jax.numpy as jnp
import numpy as np
from jax import lax
from jax.experimental import pallas as pl
from jax.experimental.pallas import tpu as pltpu
from jax.experimental.pallas import tpu_sc as plsc

N = 10000
E = 320000
DIN = 128
H = 64
R = 8
C = 32
G = 128

NC = 2
NS = 16
NW = NC * NS
L = 16

K = 128
CPW = 80
E_PER_W2 = K * CPW
E2 = NW * E_PER_W2
EPAD = E2 - E
TOTAL_CHUNKS = NW * CPW
E_PER_TILE2 = E2 // NS
CK = 2560
CNT_PAD = 81920
MROUNDS = 8
SCHUNK = CNT_PAD // MROUNDS
SSLICE = SCHUNK // NS
ACC_SLICE = N // NS

_p64 = np.zeros(H, np.int64)
for _m in range(H // 32):
    for _i in range(16):
        _p64[_m * 32 + 2 * _i] = _m * 32 + _i
        _p64[_m * 32 + 2 * _i + 1] = _m * 32 + 16 + _i
COLIDX = np.concatenate([_r * H + _p64 for _r in range(R)])


def _mesh():
    return plsc.VectorSubcoreMesh(core_axis_name="c", subcore_axis_name="s")


def _prep_body(src_hbm, dst_hbm, typ_hbm, meta_hbm, wts_hbm,
               stage_sp, cnt_sp, cnt_loc, b_s, b_d, b_t, mbuf, wbuf,
               abuf, tbuf, psem):
    cid = lax.axis_index("c")
    sid = lax.axis_index("s")
    zero16 = jnp.zeros((L,), jnp.float32)
    ones16 = jnp.full((L,), 1.0, jnp.float32)

    def zloop(i, _):
        cnt_loc[pl.ds(i * L, L)] = zero16
        return 0

    lax.fori_loop(0, CNT_PAD // L, zloop, 0)

    def count_chunk(c, _):
        base = sid * E_PER_TILE2 + c * CK
        pltpu.sync_copy(dst_hbm.at[pl.ds(base, CK)], b_d)
        pltpu.sync_copy(typ_hbm.at[pl.ds(base, CK)], b_t)

        def inner(j, _):
            d = b_d[pl.ds(j * L, L)]
            t = b_t[pl.ds(j * L, L)]
            plsc.addupdate_scatter(cnt_loc, [d * R + t], ones16)
            return 0

        lax.fori_loop(0, CK // L, inner, 0)
        return 0

    lax.fori_loop(0, E_PER_TILE2 // CK, count_chunk, 0)

    for rr in range(MROUNDS):
        pltpu.sync_copy(cnt_loc.at[pl.ds(rr * SCHUNK, SCHUNK)],
                        stage_sp.at[sid])
        plsc.subcore_barrier()
        descs = [
            pltpu.async_copy(
                stage_sp.at[o, pl.ds(sid * SSLICE, SSLICE)],
                tbuf.at[o], psem)
            for o in range(NS)
        ]
        for dsc in descs:
            dsc.wait()

        def acc_slice(i, _):
            tot = tbuf[0, pl.ds(i * L, L)]
            for o in range(1, NS):
                tot = tot + tbuf[o, pl.ds(i * L, L)]
            abuf[pl.ds(i * L, L)] = tot
            return 0

        lax.fori_loop(0, SSLICE // L, acc_slice, 0)
        pltpu.sync_copy(
            abuf, cnt_sp.at[pl.ds(rr * SCHUNK + sid * SSLICE, SSLICE)])
        plsc.subcore_barrier()
    pltpu.sync_copy(cnt_sp, cnt_loc)

    wid = sid * NC + cid

    def metachunk(it, _):
        base = wid * E_PER_W2 + it * CK
        pltpu.sync_copy(src_hbm.at[pl.ds(base, CK)], b_s)
        pltpu.sync_copy(dst_hbm.at[pl.ds(base, CK)], b_d)
        pltpu.sync_copy(typ_hbm.at[pl.ds(base, CK)], b_t)

        for q in range(CK // K):
            def inner(j2, _, q=q):
                j = q * (K // L) + j2
                r16 = j2 * L
                s = b_s[pl.ds(j * L, L)]
                d = b_d[pl.ds(j * L, L)]
                t = b_t[pl.ds(j * L, L)]
                eids = base + j * L + lax.iota(jnp.int32, L)
                real = eids < E
                cval = plsc.load_gather(cnt_loc, [d * R + t])
                wbuf[pl.ds(j * L, L)] = jnp.where(
                    real, 1.0 / jnp.maximum(cval, 1.0), 0.0)
                mbuf[pl.ds((q * 2 + 0) * K + r16, L)] = s * R + t
                mbuf[pl.ds((q * 2 + 1) * K + r16, L)] = jnp.where(real, d, 0)
                return 0

            lax.fori_loop(0, K // L, inner, 0)
        mwords = (CK // K) * 2 * K
        pltpu.sync_copy(
            mbuf, meta_hbm.at[pl.ds((wid * CPW + it * (CK // K)) * 2 * K,
                                    mwords)])
        pltpu.sync_copy(wbuf, wts_hbm.at[pl.ds(base, CK)])
        return 0

    lax.fori_loop(0, E_PER_W2 // CK, metachunk, 0)


def _run_prep(src, dst, typ):
    fn = pl.kernel(
        _prep_body,
        out_type=(jax.ShapeDtypeStruct((TOTAL_CHUNKS * 2 * K,), jnp.int32),
                  jax.ShapeDtypeStruct((E2,), jnp.float32)),
        mesh=_mesh(),
        scratch_types=[
            pltpu.VMEM_SHARED((NS, SCHUNK), jnp.float32),
            pltpu.VMEM_SHARED((CNT_PAD,), jnp.float32),
            pltpu.VMEM((CNT_PAD,), jnp.float32),
            pltpu.VMEM((CK,), jnp.int32),
            pltpu.VMEM((CK,), jnp.int32),
            pltpu.VMEM((CK,), jnp.int32),
            pltpu.VMEM(((CK // K) * 2 * K,), jnp.int32),
            pltpu.VMEM((CK,), jnp.float32),
            pltpu.VMEM((SSLICE,), jnp.float32),
            pltpu.VMEM((NS, SSLICE), jnp.float32),
            pltpu.SemaphoreType.DMA,
        ],
        compiler_params=pltpu.CompilerParams(needs_layout_passes=False),
    )
    return fn(src, dst, typ)


def _agg_body(hr_hbm, meta_hbm, wts_hbm, out_hbm, acc_sp, mall, wall,
              i0, i1, i2, i3, r0, r1, r2, r3, d0, d1, d2, d3, zb,
              g0, g1, g2, g3, s0, s1, s2, s3):
    rins = [i0, i1, i2, i3]
    rows = [r0, r1, r2, r3]
    dbs = [d0, d1, d2, d3]
    gs = [g0, g1, g2, g3]
    ss = [s0, s1, s2, s3]
    cid = lax.axis_index("c")
    sid = lax.axis_index("s")
    wid = sid * NC + cid
    zero16 = jnp.zeros((L,), jnp.float32)

    def zloop(a, _):
        for j in range(H // L):
            zb[a, pl.ds(j * L, L)] = zero16
        return 0

    lax.fori_loop(0, 125, zloop, 0)
    for v in range(ACC_SLICE // 125):
        pltpu.sync_copy(zb, acc_sp.at[pl.ds(sid * ACC_SLICE + v * 125, 125)])
    plsc.subcore_barrier()

    mwords = CPW * 2 * K
    pltpu.sync_copy(meta_hbm.at[pl.ds(wid * mwords, mwords)], mall)
    pltpu.sync_copy(wts_hbm.at[pl.ds(wid * E_PER_W2, E_PER_W2)], wall)

    def pipe(i, _):
        for b in range(4):
            kk = i * 4 + b

            @pl.when(kk < CPW)
            def _():
                @pl.when(kk >= 4)
                def _():
                    pltpu.make_async_copy(
                        rows[b], acc_sp.at[dbs[b]], ss[b]).wait()
                pltpu.async_copy(
                    hr_hbm.at[mall.at[pl.ds(kk * 2 * K, K)]], rins[b], gs[b])

            jj = kk - 3
            bj = (b + 1) % 4

            @pl.when(jnp.logical_and(jj >= 0, jj < CPW))
            def _():
                pltpu.make_async_copy(
                    hr_hbm.at[mall.at[pl.ds(jj * 2 * K, K)]], rins[bj],
                    gs[bj]).wait()

                def scale(q, _):
                    wv = wall[pl.ds(jj * K + q * L, L)]
                    dbs[bj][pl.ds(q * L, L)] = (
                        mall[pl.ds((jj * 2 + 1) * K + q * L, L)])
                    for l in range(L):
                        w = wv[l]
                        e = q * L + l
                        for m in range(H // 32):
                            v = rins[bj][e, pl.ds(m * 32, 32)]
                            lo, hi = plsc.unpack(
                                v, format=plsc.PackFormat.INTERLEAVED)
                            rows[bj][e, pl.ds(m * 32, L)] = lo * w
                            rows[bj][e, pl.ds(m * 32 + L, L)] = hi * w
                    return 0

                lax.fori_loop(0, K // L, scale, 0)
                pltpu.async_copy(rows[bj], acc_sp.at[dbs[bj]],
                                 ss[bj], add=True)
        return 0

    lax.fori_loop(0, (CPW + 3 + 3) // 4 + 1, pipe, 0)
    for b in range(4):
        pltpu.make_async_copy(rows[b], acc_sp.at[dbs[b]], ss[b]).wait()
    plsc.subcore_barrier()
    pltpu.sync_copy(acc_sp.at[pl.ds(sid * ACC_SLICE, ACC_SLICE)],
                    out_hbm.at[cid, pl.ds(sid * ACC_SLICE, ACC_SLICE)])


def _run_agg(hr, meta, wts):
    fn = pl.kernel(
        _agg_body,
        out_type=jax.ShapeDtypeStruct((NC, N, H), jnp.float32),
        mesh=_mesh(),
        scratch_types=[
            pltpu.VMEM_SHARED((N, H), jnp.float32),
            pltpu.VMEM((CPW * 2 * K,), jnp.int32),
            pltpu.VMEM((E_PER_W2,), jnp.float32),
            pltpu.VMEM((K, H), jnp.bfloat16),
            pltpu.VMEM((K, H), jnp.bfloat16),
            pltpu.VMEM((K, H), jnp.bfloat16),
            pltpu.VMEM((K, H), jnp.bfloat16),
            pltpu.VMEM((K, H), jnp.float32),
            pltpu.VMEM((K, H), jnp.float32),
            pltpu.VMEM((K, H), jnp.float32),
            pltpu.VMEM((K, H), jnp.float32),
            pltpu.VMEM((K,), jnp.int32),
            pltpu.VMEM((K,), jnp.int32),
            pltpu.VMEM((K,), jnp.int32),
            pltpu.VMEM((K,), jnp.int32),
            pltpu.VMEM((125, H), jnp.float32),
            pltpu.SemaphoreType.DMA,
            pltpu.SemaphoreType.DMA,
            pltpu.SemaphoreType.DMA,
            pltpu.SemaphoreType.DMA,
            pltpu.SemaphoreType.DMA,
            pltpu.SemaphoreType.DMA,
            pltpu.SemaphoreType.DMA,
            pltpu.SemaphoreType.DMA,
        ],
        compiler_params=pltpu.CompilerParams(use_tc_tiling_on_sc=False,
                                             needs_layout_passes=False),
    )
    return fn(hr, meta, wts)


NB = 5
BN = N // NB


def _dot(a, b):
    return jnp.dot(a, b, preferred_element_type=jnp.float32)


def _enc_kernel(x_ref, ew_ref, eb_ref, wcat_ref, h0_ref, hr1_ref):
    h0 = _dot(x_ref[...], ew_ref[...]) + eb_ref[...]
    h0_ref[...] = h0
    hr1_ref[...] = _dot(h0, wcat_ref[...]).astype(jnp.bfloat16)


def _run_encoder(x, enc_W, enc_b2, wcat1):
    return pl.pallas_call(
        _enc_kernel,
        grid=(NB,),
        in_specs=[
            pl.BlockSpec((BN, DIN), lambda i: (i, 0)),
            pl.BlockSpec((DIN, H), lambda i: (0, 0)),
            pl.BlockSpec((1, H), lambda i: (0, 0)),
            pl.BlockSpec((H, R * H), lambda i: (0, 0)),
        ],
        out_specs=[
            pl.BlockSpec((BN, H), lambda i: (i, 0)),
            pl.BlockSpec((BN, R * H), lambda i: (i, 0)),
        ],
        out_shape=[
            jax.ShapeDtypeStruct((N, H), jnp.float32),
            jax.ShapeDtypeStruct((N, R * H), jnp.bfloat16),
        ],
    )(x, enc_W, enc_b2, wcat1)


def _epi1_kernel(acc_ref, h_ref, wr_ref, b_ref, wcat_ref, h1_ref, hr2_ref):
    o = acc_ref[0] + acc_ref[1] + _dot(h_ref[...], wr_ref[...]) + b_ref[...]
    o = jnp.maximum(o, 0.0)
    h1_ref[...] = o
    hr2_ref[...] = _dot(o, wcat_ref[...]).astype(jnp.bfloat16)


def _run_epi1(accs, h, W_root, b2, wcat2):
    return pl.pallas_call(
        _epi1_kernel,
        grid=(NB,),
        in_specs=[
            pl.BlockSpec((NC, BN, H), lambda i: (0, i, 0)),
            pl.BlockSpec((BN, H), lambda i: (i, 0)),
            pl.BlockSpec((H, H), lambda i: (0, 0)),
            pl.BlockSpec((1, H), lambda i: (0, 0)),
            pl.BlockSpec((H, R * H), lambda i: (0, 0)),
        ],
        out_specs=[
            pl.BlockSpec((BN, H), lambda i: (i, 0)),
            pl.BlockSpec((BN, R * H), lambda i: (i, 0)),
        ],
        out_shape=[
            jax.ShapeDtypeStruct((N, H), jnp.float32),
            jax.ShapeDtypeStruct((N, R * H), jnp.bfloat16),
        ],
    )(accs, h, W_root, b2, wcat2)


def _epi2_kernel(acc_ref, h_ref, wr_ref, b_ref, h2_ref):
    h2_ref[...] = (acc_ref[0] + acc_ref[1]
                   + _dot(h_ref[...], wr_ref[...]) + b_ref[...])


def _run_epi2(accs, h, W_root, b2):
    return pl.pallas_call(
        _epi2_kernel,
        grid=(NB,),
        in_specs=[
            pl.BlockSpec((NC, BN, H), lambda i: (0, i, 0)),
            pl.BlockSpec((BN, H), lambda i: (i, 0)),
            pl.BlockSpec((H, H), lambda i: (0, 0)),
            pl.BlockSpec((1, H), lambda i: (0, 0)),
        ],
        out_specs=pl.BlockSpec((BN, H), lambda i: (i, 0)),
        out_shape=jax.ShapeDtypeStruct((N, H), jnp.float32),
    )(accs, h, W_root, b2)


def _pool_kernel(h_ref, batch_ref, lw_ref, lb_ref, cw_ref, cb_ref,
                 out_ref, pool_ref):
    i = pl.program_id(0)

    @pl.when(i == 0)
    def _():
        pool_ref[...] = jnp.zeros_like(pool_ref)

    ids = batch_ref[0]
    iota = lax.broadcasted_iota(jnp.int32, (G, BN), 0)
    onehot = (iota == ids).astype(jnp.float32)
    pool_ref[...] += _dot(onehot, h_ref[...])

    @pl.when(i == NB - 1)
    def _():
        z = jnp.maximum(_dot(pool_ref[...], lw_ref[...]) + lb_ref[...], 0.0)
        out_ref[...] = _dot(z, cw_ref[...]) + cb_ref[...]


def _run_pool(h2, batch3, lin_W, lin_b2, clf_W, clf_b2):
    return pl.pallas_call(
        _pool_kernel,
        grid=(NB,),
        in_specs=[
            pl.BlockSpec((BN, H), lambda i: (i, 0)),
            pl.BlockSpec((1, 1, BN), lambda i: (i, 0, 0)),
            pl.BlockSpec((H, H), lambda i: (0, 0)),
            pl.BlockSpec((1, H), lambda i: (0, 0)),
            pl.BlockSpec((H, C), lambda i: (0, 0)),
            pl.BlockSpec((1, C), lambda i: (0, 0)),
        ],
        out_specs=pl.BlockSpec((G, C), lambda i: (0, 0)),
        out_shape=jax.ShapeDtypeStruct((G, C), jnp.float32),
        scratch_shapes=[pltpu.VMEM((G, H), jnp.float32)],
    )(h2, batch3, lin_W, lin_b2, clf_W, clf_b2)


def kernel(x, edge_index, edge_type, batch, enc_W, enc_b, W_rel1, W_root1,
           b1, W_rel2, W_root2, b2, lin_W, lin_b, clf_W, clf_b):
    src = jnp.concatenate([edge_index[0],
                           jnp.zeros((EPAD,), jnp.int32)])
    dst = jnp.concatenate([edge_index[1],
                           jnp.full((EPAD,), N, jnp.int32)])
    typ = jnp.concatenate([edge_type, jnp.zeros((EPAD,), jnp.int32)])

    wcat1 = jnp.transpose(W_rel1, (1, 0, 2)).reshape(H, R * H)[:, COLIDX]
    wcat2 = jnp.transpose(W_rel2, (1, 0, 2)).reshape(H, R * H)[:, COLIDX]

    meta, wts = _run_prep(src, dst, typ)

    h0, hr1 = _run_encoder(x, enc_W, enc_b.reshape(1, H), wcat1)
    accs1 = _run_agg(hr1.reshape(N * R, H), meta, wts)
    h1, hr2 = _run_epi1(accs1, h0, W_root1, b1.reshape(1, H), wcat2)
    accs2 = _run_agg(hr2.reshape(N * R, H), meta, wts)
    h2 = _run_epi2(accs2, h1, W_root2, b2.reshape(1, H))

    batch3 = batch.reshape(NB, 1, BN)
    return _run_pool(h2, batch3, lin_W, lin_b.reshape(1, H),
                     clf_W, clf_b.reshape(1, C))


# speedup vs baseline: 1.1000x; 1.1000x over previous
import jax
import jax.numpy as jnp
import numpy as np
from jax import lax
from jax.experimental import pallas as pl
from jax.experimental.pallas import tpu as pltpu
from jax.experimental.pallas import tpu_sc as plsc

N = 10000
E = 320000
DIN = 128
H = 64
R = 8
C = 32
G = 128

NC = 2
NS = 16
NW = NC * NS
L = 16

K = 128
CPW = 80
E_PER_W2 = K * CPW
E2 = NW * E_PER_W2
EPAD = E2 - E
TOTAL_CHUNKS = NW * CPW
E_PER_TILE2 = E2 // NS
CK = 2560
CNT_PAD = 81920
MROUNDS = 8
SCHUNK = CNT_PAD // MROUNDS
SSLICE = SCHUNK // NS
ACC_SLICE = N // NS

_p64 = np.zeros(H, np.int64)
for _m in range(H // 32):
    for _i in range(16):
        _p64[_m * 32 + 2 * _i] = _m * 32 + _i
        _p64[_m * 32 + 2 * _i + 1] = _m * 32 + 16 + _i
COLIDX = np.concatenate([_r * H + _p64 for _r in range(R)])


def _mesh():
    return plsc.VectorSubcoreMesh(core_axis_name="c", subcore_axis_name="s")


def _prep_body(src_hbm, dst_hbm, typ_hbm, meta_hbm, wts_hbm,
               stage_sp, cnt_sp, cnt_loc, b_s, b_d, b_t, mbuf, wbuf,
               abuf, tbuf, psem):
    cid = lax.axis_index("c")
    sid = lax.axis_index("s")
    zero16 = jnp.zeros((L,), jnp.float32)
    ones16 = jnp.full((L,), 1.0, jnp.float32)

    def zloop(i, _):
        cnt_loc[pl.ds(i * L, L)] = zero16
        return 0

    lax.fori_loop(0, CNT_PAD // L, zloop, 0)

    def count_chunk(c, _):
        base = sid * E_PER_TILE2 + c * CK
        pltpu.sync_copy(dst_hbm.at[pl.ds(base, CK)], b_d)
        pltpu.sync_copy(typ_hbm.at[pl.ds(base, CK)], b_t)

        def inner(j, _):
            d = b_d[pl.ds(j * L, L)]
            t = b_t[pl.ds(j * L, L)]
            plsc.addupdate_scatter(cnt_loc, [d * R + t], ones16)
            return 0

        lax.fori_loop(0, CK // L, inner, 0)
        return 0

    lax.fori_loop(0, E_PER_TILE2 // CK, count_chunk, 0)

    for rr in range(MROUNDS):
        pltpu.sync_copy(cnt_loc.at[pl.ds(rr * SCHUNK, SCHUNK)],
                        stage_sp.at[sid])
        plsc.subcore_barrier()
        descs = [
            pltpu.async_copy(
                stage_sp.at[o, pl.ds(sid * SSLICE, SSLICE)],
                tbuf.at[o], psem)
            for o in range(NS)
        ]
        for dsc in descs:
            dsc.wait()

        def acc_slice(i, _):
            tot = tbuf[0, pl.ds(i * L, L)]
            for o in range(1, NS):
                tot = tot + tbuf[o, pl.ds(i * L, L)]
            abuf[pl.ds(i * L, L)] = tot
            return 0

        lax.fori_loop(0, SSLICE // L, acc_slice, 0)
        pltpu.sync_copy(
            abuf, cnt_sp.at[pl.ds(rr * SCHUNK + sid * SSLICE, SSLICE)])
        plsc.subcore_barrier()
    pltpu.sync_copy(cnt_sp, cnt_loc)

    wid = sid * NC + cid

    def metachunk(it, _):
        base = wid * E_PER_W2 + it * CK
        pltpu.sync_copy(src_hbm.at[pl.ds(base, CK)], b_s)
        pltpu.sync_copy(dst_hbm.at[pl.ds(base, CK)], b_d)
        pltpu.sync_copy(typ_hbm.at[pl.ds(base, CK)], b_t)

        for q in range(CK // K):
            def inner(j2, _, q=q):
                j = q * (K // L) + j2
                r16 = j2 * L
                s = b_s[pl.ds(j * L, L)]
                d = b_d[pl.ds(j * L, L)]
                t = b_t[pl.ds(j * L, L)]
                eids = base + j * L + lax.iota(jnp.int32, L)
                real = eids < E
                cval = plsc.load_gather(cnt_loc, [d * R + t])
                wbuf[pl.ds(j * L, L)] = jnp.where(
                    real, 1.0 / jnp.maximum(cval, 1.0), 0.0)
                mbuf[pl.ds((q * 2 + 0) * K + r16, L)] = s * R + t
                mbuf[pl.ds((q * 2 + 1) * K + r16, L)] = jnp.where(real, d, 0)
                return 0

            lax.fori_loop(0, K // L, inner, 0)
        mwords = (CK // K) * 2 * K
        pltpu.sync_copy(
            mbuf, meta_hbm.at[pl.ds((wid * CPW + it * (CK // K)) * 2 * K,
                                    mwords)])
        pltpu.sync_copy(wbuf, wts_hbm.at[pl.ds(base, CK)])
        return 0

    lax.fori_loop(0, E_PER_W2 // CK, metachunk, 0)


def _run_prep(src, dst, typ):
    fn = pl.kernel(
        _prep_body,
        out_type=(jax.ShapeDtypeStruct((TOTAL_CHUNKS * 2 * K,), jnp.int32),
                  jax.ShapeDtypeStruct((E2,), jnp.float32)),
        mesh=_mesh(),
        scratch_types=[
            pltpu.VMEM_SHARED((NS, SCHUNK), jnp.float32),
            pltpu.VMEM_SHARED((CNT_PAD,), jnp.float32),
            pltpu.VMEM((CNT_PAD,), jnp.float32),
            pltpu.VMEM((CK,), jnp.int32),
            pltpu.VMEM((CK,), jnp.int32),
            pltpu.VMEM((CK,), jnp.int32),
            pltpu.VMEM(((CK // K) * 2 * K,), jnp.int32),
            pltpu.VMEM((CK,), jnp.float32),
            pltpu.VMEM((SSLICE,), jnp.float32),
            pltpu.VMEM((NS, SSLICE), jnp.float32),
            pltpu.SemaphoreType.DMA,
        ],
        compiler_params=pltpu.CompilerParams(needs_layout_passes=False),
    )
    return fn(src, dst, typ)


def _agg_body(hr_hbm, meta_hbm, wts_hbm, out_hbm, acc_sp, mall, wall,
              i0, i1, i2, i3, r0, r1, r2, r3, d0, d1, d2, d3, zb,
              g0, g1, g2, g3, s0, s1, s2, s3):
    rins = [i0, i1, i2, i3]
    rows = [r0, r1, r2, r3]
    dbs = [d0, d1, d2, d3]
    gs = [g0, g1, g2, g3]
    ss = [s0, s1, s2, s3]
    cid = lax.axis_index("c")
    sid = lax.axis_index("s")
    wid = sid * NC + cid
    zero16 = jnp.zeros((L,), jnp.float32)

    def zloop(a, _):
        for j in range(H // L):
            zb[a, pl.ds(j * L, L)] = zero16
        return 0

    lax.fori_loop(0, 125, zloop, 0)
    for v in range(ACC_SLICE // 125):
        pltpu.sync_copy(zb, acc_sp.at[pl.ds(sid * ACC_SLICE + v * 125, 125)])
    plsc.subcore_barrier()

    mwords = CPW * 2 * K
    pltpu.sync_copy(meta_hbm.at[pl.ds(wid * mwords, mwords)], mall)
    pltpu.sync_copy(wts_hbm.at[pl.ds(wid * E_PER_W2, E_PER_W2)], wall)

    def pipe(i, _):
        for b in range(4):
            kk = i * 4 + b

            @pl.when(kk < CPW)
            def _():
                @pl.when(kk >= 4)
                def _():
                    pltpu.make_async_copy(
                        rows[b], acc_sp.at[dbs[b]], ss[b]).wait()
                pltpu.async_copy(
                    hr_hbm.at[mall.at[pl.ds(kk * 2 * K, K)]], rins[b], gs[b])

            jj = kk - 2
            bj = (b + 2) % 4

            @pl.when(jnp.logical_and(jj >= 0, jj < CPW))
            def _():
                pltpu.make_async_copy(
                    hr_hbm.at[mall.at[pl.ds(jj * 2 * K, K)]], rins[bj],
                    gs[bj]).wait()

                def scale(q, _):
                    wv = wall[pl.ds(jj * K + q * L, L)]
                    dbs[bj][pl.ds(q * L, L)] = (
                        mall[pl.ds((jj * 2 + 1) * K + q * L, L)])
                    for l in range(L):
                        w = wv[l]
                        e = q * L + l
                        for m in range(H // 32):
                            v = rins[bj][e, pl.ds(m * 32, 32)]
                            lo, hi = plsc.unpack(
                                v, format=plsc.PackFormat.INTERLEAVED)
                            rows[bj][e, pl.ds(m * 32, L)] = lo * w
                            rows[bj][e, pl.ds(m * 32 + L, L)] = hi * w
                    return 0

                lax.fori_loop(0, K // L, scale, 0)
                pltpu.async_copy(rows[bj], acc_sp.at[dbs[bj]],
                                 ss[bj], add=True)
        return 0

    lax.fori_loop(0, (CPW + 2 + 3) // 4 + 1, pipe, 0)
    for b in range(4):
        pltpu.make_async_copy(rows[b], acc_sp.at[dbs[b]], ss[b]).wait()
    plsc.subcore_barrier()
    pltpu.sync_copy(acc_sp.at[pl.ds(sid * ACC_SLICE, ACC_SLICE)],
                    out_hbm.at[cid, pl.ds(sid * ACC_SLICE, ACC_SLICE)])


def _run_agg(hr, meta, wts):
    fn = pl.kernel(
        _agg_body,
        out_type=jax.ShapeDtypeStruct((NC, N, H), jnp.float32),
        mesh=_mesh(),
        scratch_types=[
            pltpu.VMEM_SHARED((N, H), jnp.float32),
            pltpu.VMEM((CPW * 2 * K,), jnp.int32),
            pltpu.VMEM((E_PER_W2,), jnp.float32),
            pltpu.VMEM((K, H), jnp.bfloat16),
            pltpu.VMEM((K, H), jnp.bfloat16),
            pltpu.VMEM((K, H), jnp.bfloat16),
            pltpu.VMEM((K, H), jnp.bfloat16),
            pltpu.VMEM((K, H), jnp.float32),
            pltpu.VMEM((K, H), jnp.float32),
            pltpu.VMEM((K, H), jnp.float32),
            pltpu.VMEM((K, H), jnp.float32),
            pltpu.VMEM((K,), jnp.int32),
            pltpu.VMEM((K,), jnp.int32),
            pltpu.VMEM((K,), jnp.int32),
            pltpu.VMEM((K,), jnp.int32),
            pltpu.VMEM((125, H), jnp.float32),
            pltpu.SemaphoreType.DMA,
            pltpu.SemaphoreType.DMA,
            pltpu.SemaphoreType.DMA,
            pltpu.SemaphoreType.DMA,
            pltpu.SemaphoreType.DMA,
            pltpu.SemaphoreType.DMA,
            pltpu.SemaphoreType.DMA,
            pltpu.SemaphoreType.DMA,
        ],
        compiler_params=pltpu.CompilerParams(use_tc_tiling_on_sc=False,
                                             needs_layout_passes=False),
    )
    return fn(hr, meta, wts)


NB = 5
BN = N // NB


def _dot(a, b):
    return jnp.dot(a, b, preferred_element_type=jnp.float32)


def _enc_kernel(x_ref, ew_ref, eb_ref, wcat_ref, h0_ref, hr1_ref):
    h0 = _dot(x_ref[...], ew_ref[...]) + eb_ref[...]
    h0_ref[...] = h0
    hr1_ref[...] = _dot(h0, wcat_ref[...]).astype(jnp.bfloat16)


def _run_encoder(x, enc_W, enc_b2, wcat1):
    return pl.pallas_call(
        _enc_kernel,
        grid=(NB,),
        in_specs=[
            pl.BlockSpec((BN, DIN), lambda i: (i, 0)),
            pl.BlockSpec((DIN, H), lambda i: (0, 0)),
            pl.BlockSpec((1, H), lambda i: (0, 0)),
            pl.BlockSpec((H, R * H), lambda i: (0, 0)),
        ],
        out_specs=[
            pl.BlockSpec((BN, H), lambda i: (i, 0)),
            pl.BlockSpec((BN, R * H), lambda i: (i, 0)),
        ],
        out_shape=[
            jax.ShapeDtypeStruct((N, H), jnp.float32),
            jax.ShapeDtypeStruct((N, R * H), jnp.bfloat16),
        ],
    )(x, enc_W, enc_b2, wcat1)


def _epi1_kernel(acc_ref, h_ref, wr_ref, b_ref, wcat_ref, h1_ref, hr2_ref):
    o = acc_ref[0] + acc_ref[1] + _dot(h_ref[...], wr_ref[...]) + b_ref[...]
    o = jnp.maximum(o, 0.0)
    h1_ref[...] = o
    hr2_ref[...] = _dot(o, wcat_ref[...]).astype(jnp.bfloat16)


def _run_epi1(accs, h, W_root, b2, wcat2):
    return pl.pallas_call(
        _epi1_kernel,
        grid=(NB,),
        in_specs=[
            pl.BlockSpec((NC, BN, H), lambda i: (0, i, 0)),
            pl.BlockSpec((BN, H), lambda i: (i, 0)),
            pl.BlockSpec((H, H), lambda i: (0, 0)),
            pl.BlockSpec((1, H), lambda i: (0, 0)),
            pl.BlockSpec((H, R * H), lambda i: (0, 0)),
        ],
        out_specs=[
            pl.BlockSpec((BN, H), lambda i: (i, 0)),
            pl.BlockSpec((BN, R * H), lambda i: (i, 0)),
        ],
        out_shape=[
            jax.ShapeDtypeStruct((N, H), jnp.float32),
            jax.ShapeDtypeStruct((N, R * H), jnp.bfloat16),
        ],
    )(accs, h, W_root, b2, wcat2)


def _epi2_kernel(acc_ref, h_ref, wr_ref, b_ref, h2_ref):
    h2_ref[...] = (acc_ref[0] + acc_ref[1]
                   + _dot(h_ref[...], wr_ref[...]) + b_ref[...])


def _run_epi2(accs, h, W_root, b2):
    return pl.pallas_call(
        _epi2_kernel,
        grid=(NB,),
        in_specs=[
            pl.BlockSpec((NC, BN, H), lambda i: (0, i, 0)),
            pl.BlockSpec((BN, H), lambda i: (i, 0)),
            pl.BlockSpec((H, H), lambda i: (0, 0)),
            pl.BlockSpec((1, H), lambda i: (0, 0)),
        ],
        out_specs=pl.BlockSpec((BN, H), lambda i: (i, 0)),
        out_shape=jax.ShapeDtypeStruct((N, H), jnp.float32),
    )(accs, h, W_root, b2)


def _pool_kernel(h_ref, batch_ref, lw_ref, lb_ref, cw_ref, cb_ref,
                 out_ref, pool_ref):
    i = pl.program_id(0)

    @pl.when(i == 0)
    def _():
        pool_ref[...] = jnp.zeros_like(pool_ref)

    ids = batch_ref[0]
    iota = lax.broadcasted_iota(jnp.int32, (G, BN), 0)
    onehot = (iota == ids).astype(jnp.float32)
    pool_ref[...] += _dot(onehot, h_ref[...])

    @pl.when(i == NB - 1)
    def _():
        z = jnp.maximum(_dot(pool_ref[...], lw_ref[...]) + lb_ref[...], 0.0)
        out_ref[...] = _dot(z, cw_ref[...]) + cb_ref[...]


def _run_pool(h2, batch3, lin_W, lin_b2, clf_W, clf_b2):
    return pl.pallas_call(
        _pool_kernel,
        grid=(NB,),
        in_specs=[
            pl.BlockSpec((BN, H), lambda i: (i, 0)),
            pl.BlockSpec((1, 1, BN), lambda i: (i, 0, 0)),
            pl.BlockSpec((H, H), lambda i: (0, 0)),
            pl.BlockSpec((1, H), lambda i: (0, 0)),
            pl.BlockSpec((H, C), lambda i: (0, 0)),
            pl.BlockSpec((1, C), lambda i: (0, 0)),
        ],
        out_specs=pl.BlockSpec((G, C), lambda i: (0, 0)),
        out_shape=jax.ShapeDtypeStruct((G, C), jnp.float32),
        scratch_shapes=[pltpu.VMEM((G, H), jnp.float32)],
    )(h2, batch3, lin_W, lin_b2, clf_W, clf_b2)


def kernel(x, edge_index, edge_type, batch, enc_W, enc_b, W_rel1, W_root1,
           b1, W_rel2, W_root2, b2, lin_W, lin_b, clf_W, clf_b):
    src = jnp.concatenate([edge_index[0],
                           jnp.zeros((EPAD,), jnp.int32)])
    dst = jnp.concatenate([edge_index[1],
                           jnp.full((EPAD,), N, jnp.int32)])
    typ = jnp.concatenate([edge_type, jnp.zeros((EPAD,), jnp.int32)])

    wcat1 = jnp.transpose(W_rel1, (1, 0, 2)).reshape(H, R * H)[:, COLIDX]
    wcat2 = jnp.transpose(W_rel2, (1, 0, 2)).reshape(H, R * H)[:, COLIDX]

    meta, wts = _run_prep(src, dst, typ)

    h0, hr1 = _run_encoder(x, enc_W, enc_b.reshape(1, H), wcat1)
    accs1 = _run_agg(hr1.reshape(N * R, H), meta, wts)
    h1, hr2 = _run_epi1(accs1, h0, W_root1, b1.reshape(1, H), wcat2)
    accs2 = _run_agg(hr2.reshape(N * R, H), meta, wts)
    h2 = _run_epi2(accs2, h1, W_root2, b2.reshape(1, H))

    batch3 = batch.reshape(NB, 1, BN)
    return _run_pool(h2, batch3, lin_W, lin_b.reshape(1, H),
                     clf_W, clf_b.reshape(1, C))
